# baseline (speedup 1.0000x reference)
import functools

import jax
import jax.numpy as jnp
import numpy as np
from jax.experimental import pallas as pl
from jax.experimental.pallas import tpu as pltpu

_HI = np.int32(-65536)


def _pack_pair(c_lo, c_hi):
    lo = pltpu.bitcast(c_lo.astype(jnp.bfloat16).astype(jnp.float32), jnp.uint32)
    hi = pltpu.bitcast(c_hi.astype(jnp.bfloat16).astype(jnp.float32), jnp.int32)
    lo_s = pltpu.bitcast(
        jax.lax.shift_right_logical(lo, np.uint32(16)), jnp.int32)
    return lo_s | (hi & _HI)


def _k0_pack_table(w1_any, top_ref, tail_ref, out_ref, bot, sems,
                   *, R, S, P, DIN, PPB):
    k = pl.program_id(0)
    cur = jax.lax.rem(k, 3)
    W = R + 8
    maxs = (DIN - W) & ~7

    def issue(kk, slot):
        start = pl.multiple_of(jnp.minimum(kk * R + P, maxs), 8)
        pltpu.make_async_copy(
            w1_any.at[pl.ds(start, W)], bot.at[slot], sems.at[slot]).start()

    def wait(slot):
        pltpu.make_async_copy(bot.at[slot], bot.at[slot], sems.at[slot]).wait()

    @pl.when(k == 0)
    def _():
        issue(k, cur)
        issue(k + 1, jax.lax.rem(k + 1, 3))

    wait(cur)

    @pl.when(k + 2 < PPB)
    def _():
        issue(k + 2, jax.lax.rem(k + 2, 3))

    top = top_ref[...]
    for q in range(S // 2):
        u = _pack_pair(top[:, 128 * q:128 * (q + 1)],
                       top[:, 128 * (q + S // 2):128 * (q + S // 2 + 1)])
        out_ref[q:R * S:S, :] = u

    def emit_bottom(bv):
        for q in range(S // 2):
            u = _pack_pair(bv[:, 128 * q:128 * (q + 1)],
                           bv[:, 128 * (q + S // 2):128 * (q + S // 2 + 1)])
            out_ref[q + S // 2:R * S:S, :] = u

    def rollslice(buf, s):
        return pltpu.roll(buf, W - s, axis=0)[0:R, :]

    botv = bot[cur]
    shift_last = (PPB - 1) * R + (P + 1) - maxs
    i_max = P - (PPB - 1) * R

    @pl.when(k != PPB - 1)
    def _():
        emit_bottom(rollslice(botv, 1))

    @pl.when(k == PPB - 1)
    def _():
        v = rollslice(botv, shift_last)
        iota = jax.lax.broadcasted_iota(jnp.int32, (R, 1), 0)
        v = jnp.where(iota == i_max - 1, tail_ref[6:7, :], v)
        v = jnp.where(iota == i_max, tail_ref[7:8, :], v)
        emit_bottom(v)


def _k1_gather_sum(tgt_s, wtab_hbm, out_ref, land, sems, *, T, DB, NBI, NBT):
    j = pl.program_id(1)
    blk = pl.program_id(0) * NBI + j
    cur = jax.lax.rem(j, 3)

    def issue(b, slot):
        base = jnp.minimum(b, NBT - 1) * DB * T
        for di in range(DB):
            for t in range(T):
                r = tgt_s[base + di * T + t]
                pltpu.make_async_copy(
                    wtab_hbm.at[r], land.at[slot, di, t],
                    sems.at[slot, t % 2]).start()

    def wait(slot):
        half = land.at[slot].at[:, 0:T // 2]
        pltpu.make_async_copy(half, half, sems.at[slot, 0]).wait()
        half2 = land.at[slot].at[:, T // 2:T]
        pltpu.make_async_copy(half2, half2, sems.at[slot, 1]).wait()

    if NBI >= 2:
        @pl.when(j == 0)
        def _():
            issue(blk, cur)
            issue(blk + 1, jax.lax.rem(j + 1, 3))

        wait(cur)
        issue(blk + 2, jax.lax.rem(j + 2, 3))

        @pl.when(j == NBI - 1)
        def _():
            wait(jax.lax.rem(j + 1, 3))
            wait(jax.lax.rem(j + 2, 3))
    else:
        issue(blk, cur)
        wait(cur)

    for di in range(DB):
        accs = None
        for t in range(T):
            vs = [land[cur, di, t, 8 * h:8 * (h + 1), :] for h in range(2)]
            parts = []
            for v in vs:
                vu = pltpu.bitcast(v, jnp.uint32)
                lo = pltpu.bitcast(
                    jax.lax.shift_left(vu, np.uint32(16)), jnp.float32)
                hi = pltpu.bitcast(v & _HI, jnp.float32)
                parts.extend((lo, hi))
            if accs is None:
                accs = parts
            else:
                accs = [a + p for a, p in zip(accs, parts)]
        for h in range(4):
            out_ref[di, 8 * h:8 * (h + 1), :] = accs[h]


def _k2_combine(dp_s, cadj_ref, a_ref, b_ref, b1_ref, sab_hbm, out_ref,
                land1, land2, sems, *, BS, S, NBI, NBT):
    j = pl.program_id(1)
    blk = pl.program_id(0) * NBI + j
    cur = jax.lax.rem(j, 3)

    def issue(b, slot):
        base = jnp.minimum(b, NBT - 1) * BS * 2
        for i in range(BS):
            d1 = dp_s[base + 2 * i]
            d2 = dp_s[base + 2 * i + 1]
            pltpu.make_async_copy(
                sab_hbm.at[d1], land1.at[slot, i], sems.at[slot]).start()
            pltpu.make_async_copy(
                sab_hbm.at[d2], land2.at[slot, i], sems.at[slot]).start()

    def wait(slot):
        pltpu.make_async_copy(land1.at[slot], land1.at[slot], sems.at[slot]).wait()
        pltpu.make_async_copy(land2.at[slot], land2.at[slot], sems.at[slot]).wait()

    if NBI >= 2:
        @pl.when(j == 0)
        def _():
            issue(blk, cur)
            issue(blk + 1, jax.lax.rem(j + 1, 3))

        wait(cur)
        issue(blk + 2, jax.lax.rem(j + 2, 3))

        @pl.when(j == NBI - 1)
        def _():
            wait(jax.lax.rem(j + 1, 3))
            wait(jax.lax.rem(j + 2, 3))
    else:
        issue(blk, cur)
        wait(cur)

    c1 = cadj_ref[:, 0:1].reshape(BS, 1, 1)
    c2 = cadj_ref[:, 1:2].reshape(BS, 1, 1)
    a = a_ref[...]
    b = b_ref[...]
    b1 = b1_ref[...]
    l1 = land1[cur]
    l2 = land2[cur]
    x1 = l1[:, :S] + l2[:, S:] + c1 * a + c2 * b + b1
    x2 = l2[:, :S] + l1[:, S:] + c2 * a + c1 * b + b1
    out_ref[0] = jnp.maximum(x1, 0.0).astype(jnp.bfloat16)
    out_ref[1] = jnp.maximum(x2, 0.0).astype(jnp.bfloat16)


def _k3_mlp(x_ref, w2_ref, b2_ref, w3_ref, b3_ref, o_ref):
    h = jnp.dot(x_ref[...], w2_ref[...], preferred_element_type=jnp.float32)
    h = jnp.maximum(h + b2_ref[...], 0.0).astype(jnp.bfloat16)
    y = jnp.dot(h, w3_ref[...], preferred_element_type=jnp.float32)
    o_ref[...] = y + b3_ref[...]


def _pick_block(n, want):
    for cand in (want, 256, 128, 64, 32, 16, 8, 4, 2, 1):
        if cand <= want and n % cand == 0:
            return cand
    return 1


def kernel(drug_pairs, drug_targets, conc, W1, b1, W2, b2, W3, b3):
    B = drug_pairs.shape[0]
    D, T = drug_targets.shape
    DIN, H1 = W1.shape
    H2 = W2.shape[1]
    P = (DIN - 2) // 2
    S = H1 // 128

    tgt = drug_targets.astype(jnp.int32)
    eq = tgt[:, :, None] == tgt[:, None, :]
    earlier = jnp.tril(jnp.ones((T, T), jnp.bool_), k=-1)
    isdup = jnp.any(eq & earlier[None], axis=2)
    tgt_a = jnp.where(isdup, P, tgt)
    ndup = jnp.sum(isdup, axis=1).astype(jnp.float32)
    dp = drug_pairs.astype(jnp.int32)
    cadj = conc.astype(jnp.float32) - ndup[dp]

    a3 = W1[P].reshape(1, S, 128)
    brow = W1[2 * P + 1].reshape(1, S, 128)
    b1r = b1.reshape(1, S, 128)
    tgt_flat = tgt_a.reshape(D * T)
    dp_flat = dp.reshape(2 * B)

    R = 288 if (P + 1) > 288 else 8
    PPB = -(-(P + 1) // R)
    PP = PPB * R
    DB = _pick_block(D, 16)
    BS = _pick_block(B, 128)
    BS3 = _pick_block(2 * B, 512)
    NC = 1
    NB1 = D // DB // NC
    NB2 = B // BS // NC
    NB3 = 2 * B // BS3 // NC
    sem1 = ("arbitrary", "arbitrary")

    wtab2 = pl.pallas_call(
        functools.partial(_k0_pack_table, R=R, S=S, P=P, DIN=DIN, PPB=PPB),
        grid=(PPB,),
        in_specs=[
            pl.BlockSpec(memory_space=pl.ANY),
            pl.BlockSpec((R, H1), lambda k: (k, 0)),
            pl.BlockSpec((8, H1), lambda k: (0, 0)),
        ],
        out_specs=pl.BlockSpec((R * S, 128), lambda k: (k, 0)),
        out_shape=jax.ShapeDtypeStruct((PP * S, 128), jnp.int32),
        scratch_shapes=[
            pltpu.VMEM((3, R + 8, H1), jnp.float32),
            pltpu.SemaphoreType.DMA((3,)),
        ],
        compiler_params=pltpu.CompilerParams(
            dimension_semantics=("arbitrary",)),
        name="pack_table",
    )(W1, W1, W1[DIN - 8:])
    wtab = wtab2.reshape(PP, S, 128)

    sab = pl.pallas_call(
        functools.partial(_k1_gather_sum, T=T, DB=DB, NBI=NB1, NBT=D // DB),
        grid=(NC, NB1),
        in_specs=[
            pl.BlockSpec(memory_space=pltpu.SMEM),
            pl.BlockSpec(memory_space=pl.ANY),
        ],
        out_specs=pl.BlockSpec((DB, 2 * S, 128),
                               lambda c, i: (c * NB1 + i, 0, 0)),
        out_shape=jax.ShapeDtypeStruct((D, 2 * S, 128), jnp.float32),
        scratch_shapes=[
            pltpu.VMEM((3, DB, T, S, 128), jnp.int32),
            pltpu.SemaphoreType.DMA((3, 2)),
        ],
        compiler_params=pltpu.CompilerParams(
            dimension_semantics=sem1),
        name="drug_gather_sum",
    )(tgt_flat, wtab)

    xh = pl.pallas_call(
        functools.partial(_k2_combine, BS=BS, S=S, NBI=NB2, NBT=B // BS),
        grid=(NC, NB2),
        in_specs=[
            pl.BlockSpec(memory_space=pltpu.SMEM),
            pl.BlockSpec((BS, 2), lambda c, i: (c * NB2 + i, 0)),
            pl.BlockSpec((1, S, 128), lambda c, i: (0, 0, 0)),
            pl.BlockSpec((1, S, 128), lambda c, i: (0, 0, 0)),
            pl.BlockSpec((1, S, 128), lambda c, i: (0, 0, 0)),
            pl.BlockSpec(memory_space=pl.ANY),
        ],
        out_specs=pl.BlockSpec((2, BS, S, 128),
                               lambda c, i: (0, c * NB2 + i, 0, 0)),
        out_shape=jax.ShapeDtypeStruct((2, B, S, 128), jnp.bfloat16),
        scratch_shapes=[
            pltpu.VMEM((3, BS, 2 * S, 128), jnp.float32),
            pltpu.VMEM((3, BS, 2 * S, 128), jnp.float32),
            pltpu.SemaphoreType.DMA((3,)),
        ],
        compiler_params=pltpu.CompilerParams(
            dimension_semantics=sem1),
        name="combine_relu",
    )(dp_flat, cadj, a3, brow, b1r, sab)

    xall = xh.reshape(2 * B, H1)
    w2b = W2.astype(jnp.bfloat16)
    b2r = b2.reshape(1, H2)
    w3p = jnp.pad(W3, ((0, 0), (0, 127))).astype(jnp.bfloat16)
    b3p = jnp.pad(b3.reshape(1, 1), ((0, 0), (0, 127)))

    y2 = pl.pallas_call(
        _k3_mlp,
        grid=(NC, NB3),
        in_specs=[
            pl.BlockSpec((BS3, H1), lambda c, i: (c * NB3 + i, 0)),
            pl.BlockSpec((H1, H2), lambda c, i: (0, 0)),
            pl.BlockSpec((1, H2), lambda c, i: (0, 0)),
            pl.BlockSpec((H2, 128), lambda c, i: (0, 0)),
            pl.BlockSpec((1, 128), lambda c, i: (0, 0)),
        ],
        out_specs=pl.BlockSpec((BS3, 128), lambda c, i: (c * NB3 + i, 0)),
        out_shape=jax.ShapeDtypeStruct((2 * B, 128), jnp.float32),
        compiler_params=pltpu.CompilerParams(
            dimension_semantics=sem1),
        name="mlp_tail",
    )(xall, w2b, b2r, w3p, b3p)

    return (y2[0:B, 0], y2[B:2 * B, 0])


# speedup vs baseline: 1.0005x; 1.0005x over previous
import functools

import jax
import jax.numpy as jnp
import numpy as np
from jax.experimental import pallas as pl
from jax.experimental.pallas import tpu as pltpu

_HI = np.int32(-65536)


def _pack_pair(c_lo, c_hi):
    lo = pltpu.bitcast(c_lo.astype(jnp.bfloat16).astype(jnp.float32), jnp.uint32)
    hi = pltpu.bitcast(c_hi.astype(jnp.bfloat16).astype(jnp.float32), jnp.int32)
    lo_s = pltpu.bitcast(
        jax.lax.shift_right_logical(lo, np.uint32(16)), jnp.int32)
    return lo_s | (hi & _HI)


def _k0_pack_table(w1_any, top_ref, tail_ref, out_ref, bot, sems,
                   *, R, S, P, DIN, PPB):
    k = pl.program_id(0)
    cur = jax.lax.rem(k, 3)
    W = R + 8
    maxs = (DIN - W) & ~7

    def issue(kk, slot):
        start = pl.multiple_of(jnp.minimum(kk * R + P, maxs), 8)
        pltpu.make_async_copy(
            w1_any.at[pl.ds(start, W)], bot.at[slot], sems.at[slot]).start()

    def wait(slot):
        pltpu.make_async_copy(bot.at[slot], bot.at[slot], sems.at[slot]).wait()

    @pl.when(k == 0)
    def _():
        issue(k, cur)
        issue(k + 1, jax.lax.rem(k + 1, 3))

    wait(cur)

    @pl.when(k + 2 < PPB)
    def _():
        issue(k + 2, jax.lax.rem(k + 2, 3))

    top = top_ref[...]
    for q in range(S // 2):
        u = _pack_pair(top[:, 128 * q:128 * (q + 1)],
                       top[:, 128 * (q + S // 2):128 * (q + S // 2 + 1)])
        out_ref[q:R * S:S, :] = u

    def emit_bottom(bv):
        for q in range(S // 2):
            u = _pack_pair(bv[:, 128 * q:128 * (q + 1)],
                           bv[:, 128 * (q + S // 2):128 * (q + S // 2 + 1)])
            out_ref[q + S // 2:R * S:S, :] = u

    def rollslice(buf, s):
        return pltpu.roll(buf, W - s, axis=0)[0:R, :]

    botv = bot[cur]
    shift_last = (PPB - 1) * R + (P + 1) - maxs
    i_max = P - (PPB - 1) * R

    @pl.when(k != PPB - 1)
    def _():
        emit_bottom(rollslice(botv, 1))

    @pl.when(k == PPB - 1)
    def _():
        v = rollslice(botv, shift_last)
        iota = jax.lax.broadcasted_iota(jnp.int32, (R, 1), 0)
        v = jnp.where(iota == i_max - 1, tail_ref[6:7, :], v)
        v = jnp.where(iota == i_max, tail_ref[7:8, :], v)
        emit_bottom(v)


def _k1_gather_sum(tgt_s, wtab_hbm, out_ref, land, sems, *, T, DB, NBI, NBT):
    j = pl.program_id(1)
    blk = pl.program_id(0) * NBI + j
    cur = jax.lax.rem(j, 3)

    def issue(b, slot):
        base = jnp.minimum(b, NBT - 1) * DB * T
        for di in range(DB):
            for t in range(T):
                r = tgt_s[base + di * T + t]
                pltpu.make_async_copy(
                    wtab_hbm.at[r], land.at[slot, di, t], sems.at[slot]).start()

    def wait(slot):
        pltpu.make_async_copy(land.at[slot], land.at[slot], sems.at[slot]).wait()

    if NBI >= 2:
        @pl.when(j == 0)
        def _():
            issue(blk, cur)
            issue(blk + 1, jax.lax.rem(j + 1, 3))

        wait(cur)
        issue(blk + 2, jax.lax.rem(j + 2, 3))

        @pl.when(j == NBI - 1)
        def _():
            wait(jax.lax.rem(j + 1, 3))
            wait(jax.lax.rem(j + 2, 3))
    else:
        issue(blk, cur)
        wait(cur)

    for di in range(DB):
        accs = None
        for t in range(T):
            vs = [land[cur, di, t, 8 * h:8 * (h + 1), :] for h in range(2)]
            parts = []
            for v in vs:
                vu = pltpu.bitcast(v, jnp.uint32)
                lo = pltpu.bitcast(
                    jax.lax.shift_left(vu, np.uint32(16)), jnp.float32)
                hi = pltpu.bitcast(v & _HI, jnp.float32)
                parts.extend((lo, hi))
            if accs is None:
                accs = parts
            else:
                accs = [a + p for a, p in zip(accs, parts)]
        for h in range(4):
            out_ref[di, 8 * h:8 * (h + 1), :] = accs[h]


def _k2_combine(dp_s, cadj_ref, a_ref, b_ref, b1_ref, sab_hbm, out_ref,
                land1, land2, sems, *, BS, S, NBI, NBT):
    j = pl.program_id(1)
    blk = pl.program_id(0) * NBI + j
    cur = jax.lax.rem(j, 3)

    def issue(b, slot):
        base = jnp.minimum(b, NBT - 1) * BS * 2
        for i in range(BS):
            d1 = dp_s[base + 2 * i]
            d2 = dp_s[base + 2 * i + 1]
            pltpu.make_async_copy(
                sab_hbm.at[d1], land1.at[slot, i], sems.at[slot]).start()
            pltpu.make_async_copy(
                sab_hbm.at[d2], land2.at[slot, i], sems.at[slot]).start()

    def wait(slot):
        pltpu.make_async_copy(land1.at[slot], land1.at[slot], sems.at[slot]).wait()
        pltpu.make_async_copy(land2.at[slot], land2.at[slot], sems.at[slot]).wait()

    if NBI >= 2:
        @pl.when(j == 0)
        def _():
            issue(blk, cur)
            issue(blk + 1, jax.lax.rem(j + 1, 3))

        wait(cur)
        issue(blk + 2, jax.lax.rem(j + 2, 3))

        @pl.when(j == NBI - 1)
        def _():
            wait(jax.lax.rem(j + 1, 3))
            wait(jax.lax.rem(j + 2, 3))
    else:
        issue(blk, cur)
        wait(cur)

    c1 = cadj_ref[:, 0:1].reshape(BS, 1, 1)
    c2 = cadj_ref[:, 1:2].reshape(BS, 1, 1)
    a = a_ref[...]
    b = b_ref[...]
    b1 = b1_ref[...]
    l1 = land1[cur]
    l2 = land2[cur]
    x1 = l1[:, :S] + l2[:, S:] + c1 * a + c2 * b + b1
    x2 = l2[:, :S] + l1[:, S:] + c2 * a + c1 * b + b1
    out_ref[0] = jnp.maximum(x1, 0.0).astype(jnp.bfloat16)
    out_ref[1] = jnp.maximum(x2, 0.0).astype(jnp.bfloat16)


def _k3_mlp(x_ref, w2_ref, b2_ref, w3_ref, b3_ref, o_ref):
    h = jnp.dot(x_ref[...], w2_ref[...], preferred_element_type=jnp.float32)
    h = jnp.maximum(h + b2_ref[...], 0.0).astype(jnp.bfloat16)
    y = jnp.dot(h, w3_ref[...], preferred_element_type=jnp.float32)
    o_ref[...] = y + b3_ref[...]


def _pick_block(n, want):
    for cand in (want, 256, 128, 64, 32, 16, 8, 4, 2, 1):
        if cand <= want and n % cand == 0:
            return cand
    return 1


def kernel(drug_pairs, drug_targets, conc, W1, b1, W2, b2, W3, b3):
    B = drug_pairs.shape[0]
    D, T = drug_targets.shape
    DIN, H1 = W1.shape
    H2 = W2.shape[1]
    P = (DIN - 2) // 2
    S = H1 // 128

    tgt = drug_targets.astype(jnp.int32)
    eq = tgt[:, :, None] == tgt[:, None, :]
    earlier = jnp.tril(jnp.ones((T, T), jnp.bool_), k=-1)
    isdup = jnp.any(eq & earlier[None], axis=2)
    tgt_a = jnp.where(isdup, P, tgt)
    ndup = jnp.sum(isdup, axis=1).astype(jnp.float32)
    dp = drug_pairs.astype(jnp.int32)
    cadj = conc.astype(jnp.float32) - ndup[dp]

    a3 = W1[P].reshape(1, S, 128)
    brow = W1[2 * P + 1].reshape(1, S, 128)
    b1r = b1.reshape(1, S, 128)
    tgt_flat = tgt_a.reshape(D * T)
    dp_flat = dp.reshape(2 * B)

    R = 288 if (P + 1) > 288 else 8
    PPB = -(-(P + 1) // R)
    PP = PPB * R
    DB = _pick_block(D, 16)
    BS = _pick_block(B, 128)
    BS3 = _pick_block(2 * B, 512)
    NC = 1
    NB1 = D // DB // NC
    NB2 = B // BS // NC
    NB3 = 2 * B // BS3 // NC
    sem1 = ("arbitrary", "arbitrary")

    wtab2 = pl.pallas_call(
        functools.partial(_k0_pack_table, R=R, S=S, P=P, DIN=DIN, PPB=PPB),
        grid=(PPB,),
        in_specs=[
            pl.BlockSpec(memory_space=pl.ANY),
            pl.BlockSpec((R, H1), lambda k: (k, 0)),
            pl.BlockSpec((8, H1), lambda k: (0, 0)),
        ],
        out_specs=pl.BlockSpec((R * S, 128), lambda k: (k, 0)),
        out_shape=jax.ShapeDtypeStruct((PP * S, 128), jnp.int32),
        scratch_shapes=[
            pltpu.VMEM((3, R + 8, H1), jnp.float32),
            pltpu.SemaphoreType.DMA((3,)),
        ],
        compiler_params=pltpu.CompilerParams(
            dimension_semantics=("arbitrary",)),
        name="pack_table",
    )(W1, W1, W1[DIN - 8:])
    wtab = wtab2.reshape(PP, S, 128)

    sab = pl.pallas_call(
        functools.partial(_k1_gather_sum, T=T, DB=DB, NBI=NB1, NBT=D // DB),
        grid=(NC, NB1),
        in_specs=[
            pl.BlockSpec(memory_space=pltpu.SMEM),
            pl.BlockSpec(memory_space=pl.ANY),
        ],
        out_specs=pl.BlockSpec((DB, 2 * S, 128),
                               lambda c, i: (c * NB1 + i, 0, 0)),
        out_shape=jax.ShapeDtypeStruct((D, 2 * S, 128), jnp.float32),
        scratch_shapes=[
            pltpu.VMEM((3, DB, T, S, 128), jnp.int32),
            pltpu.SemaphoreType.DMA((3,)),
        ],
        compiler_params=pltpu.CompilerParams(
            dimension_semantics=sem1),
        name="drug_gather_sum",
    )(tgt_flat, wtab)

    xh = pl.pallas_call(
        functools.partial(_k2_combine, BS=BS, S=S, NBI=NB2, NBT=B // BS),
        grid=(NC, NB2),
        in_specs=[
            pl.BlockSpec(memory_space=pltpu.SMEM),
            pl.BlockSpec((BS, 2), lambda c, i: (c * NB2 + i, 0)),
            pl.BlockSpec((1, S, 128), lambda c, i: (0, 0, 0)),
            pl.BlockSpec((1, S, 128), lambda c, i: (0, 0, 0)),
            pl.BlockSpec((1, S, 128), lambda c, i: (0, 0, 0)),
            pl.BlockSpec(memory_space=pl.ANY),
        ],
        out_specs=pl.BlockSpec((2, BS, S, 128),
                               lambda c, i: (0, c * NB2 + i, 0, 0)),
        out_shape=jax.ShapeDtypeStruct((2, B, S, 128), jnp.bfloat16),
        scratch_shapes=[
            pltpu.VMEM((3, BS, 2 * S, 128), jnp.float32),
            pltpu.VMEM((3, BS, 2 * S, 128), jnp.float32),
            pltpu.SemaphoreType.DMA((3,)),
        ],
        compiler_params=pltpu.CompilerParams(
            dimension_semantics=sem1),
        name="combine_relu",
    )(dp_flat, cadj, a3, brow, b1r, sab)

    xall = xh.reshape(2 * B, H1)
    w2b = W2.astype(jnp.bfloat16)
    b2r = b2.reshape(1, H2)
    w3p = jnp.pad(W3, ((0, 0), (0, 127))).astype(jnp.bfloat16)
    b3p = jnp.pad(b3.reshape(1, 1), ((0, 0), (0, 127)))

    y2 = pl.pallas_call(
        _k3_mlp,
        grid=(NC, NB3),
        in_specs=[
            pl.BlockSpec((BS3, H1), lambda c, i: (c * NB3 + i, 0)),
            pl.BlockSpec((H1, H2), lambda c, i: (0, 0)),
            pl.BlockSpec((1, H2), lambda c, i: (0, 0)),
            pl.BlockSpec((H2, 128), lambda c, i: (0, 0)),
            pl.BlockSpec((1, 128), lambda c, i: (0, 0)),
        ],
        out_specs=pl.BlockSpec((BS3, 128), lambda c, i: (c * NB3 + i, 0)),
        out_shape=jax.ShapeDtypeStruct((2 * B, 128), jnp.float32),
        compiler_params=pltpu.CompilerParams(
            dimension_semantics=sem1),
        name="mlp_tail",
    )(xall, w2b, b2r, w3p, b3p)

    return (y2[0:B, 0], y2[B:2 * B, 0])
